# baseline (speedup 1.0000x reference)
import functools

import jax
import jax.numpy as jnp
from jax.experimental import pallas as pl
from jax.experimental.pallas import tpu as pltpu


def _fused_body(we_hbm, x_hbm, wg_ref, be_ref, su_ref, out_ref,
                webuf, xbuf, w4_ref, sem_we, sem_x, *, cap, t, f, kwe, c):
    d = wg_ref.shape[0]

    def we_copy(i):
        return pltpu.make_async_copy(
            we_hbm.at[i // (kwe // 2), pl.ds((i % (kwe // 2)) * f, f), :],
            webuf.at[i % 6], sem_we.at[i % 6])

    def x_copy(j):
        return pltpu.make_async_copy(
            x_hbm.at[pl.ds(j * t, t), :], xbuf.at[j % 6], sem_x.at[j % 6])

    for i in range(6):
        we_copy(i).start()
    for j in range(6):
        x_copy(j).start()

    w4_ref[0:2] = jnp.transpose(wg_ref[...])
    w4_ref[2:4] = jnp.zeros_like(w4_ref[2:4])
    bs0 = jnp.sum(be_ref[0:1, :])
    bs1 = jnp.sum(be_ref[1:2, :])

    for i in range(kwe):
        we_copy(i).wait()
        part = jnp.sum(webuf[i % 6], axis=0, keepdims=True)
        w4_ref[pl.ds(2 + i // (kwe // 2), 1)] += part
        if i + 6 < kwe:
            we_copy(i + 6).start()

    cnt = jnp.float32(0.0)
    for j in range(c):
        x_copy(j).wait()
        lt = jax.lax.dot_general(
            w4_ref[...], xbuf[j % 6],
            dimension_numbers=(((1,), (1,)), ((), ())),
            preferred_element_type=jnp.float32)
        if j + 6 < c:
            x_copy(j + 6).start()
        l0, l1, t0, t1 = lt[0:1], lt[1:2], lt[2:3], lt[3:4]
        e1 = l1 > l0
        m = e1.astype(jnp.float32)
        excl = jnp.dot(m, su_ref[...], preferred_element_type=jnp.float32)
        pos1 = excl + cnt
        slin = (j * t + jax.lax.broadcasted_iota(jnp.int32, (1, t), 1)
                ).astype(jnp.float32)
        pos0 = slin - pos1
        pos = jnp.where(e1, pos1, pos0)
        within = (pos < cap).astype(jnp.float32)
        gate = jax.nn.sigmoid(jnp.abs(l1 - l0))
        tsel = jnp.where(e1, t1 + bs1, t0 + bs0)
        out_ref[0:1, pl.ds(j * t, t)] = gate * within * tsel
        cnt = cnt + jnp.sum(m)


def _lsm_body(v_ref, out_ref):
    v = v_ref[...]
    mx = jnp.max(v, axis=1, keepdims=True)
    lse = jnp.log(jnp.sum(jnp.exp(v - mx), axis=1, keepdims=True)) + mx
    out_ref[...] = v - lse


def kernel(input, wg, We, be):
    B, SEQ, D = input.shape
    E = wg.shape[1]
    S = B * SEQ
    cap = -(-S // E)

    x = input.reshape(S, D)

    F = 512
    KWE = E * (D // F)
    T = 512
    C = S // T

    ii = jax.lax.broadcasted_iota(jnp.int32, (T, T), 0)
    jj = jax.lax.broadcasted_iota(jnp.int32, (T, T), 1)
    su = (ii < jj).astype(jnp.float32)

    val = pl.pallas_call(
        functools.partial(_fused_body, cap=float(cap), t=T, f=F,
                          kwe=KWE, c=C),
        in_specs=[
            pl.BlockSpec(memory_space=pl.ANY),
            pl.BlockSpec(memory_space=pl.ANY),
            pl.BlockSpec((D, E), lambda: (0, 0)),
            pl.BlockSpec((E, D), lambda: (0, 0)),
            pl.BlockSpec((T, T), lambda: (0, 0)),
        ],
        out_specs=pl.BlockSpec((1, S), lambda: (0, 0)),
        out_shape=jax.ShapeDtypeStruct((1, S), jnp.float32),
        scratch_shapes=[
            pltpu.VMEM((6, F, D), jnp.float32),
            pltpu.VMEM((6, T, D), jnp.float32),
            pltpu.VMEM((4, D), jnp.float32),
            pltpu.SemaphoreType.DMA((6,)),
            pltpu.SemaphoreType.DMA((6,)),
        ],
    )(We, x, wg, be, su)

    v = val.reshape(B, SEQ)

    out = pl.pallas_call(
        _lsm_body,
        in_specs=[pl.BlockSpec((B, SEQ), lambda: (0, 0))],
        out_specs=pl.BlockSpec((B, SEQ), lambda: (0, 0)),
        out_shape=jax.ShapeDtypeStruct((B, SEQ), jnp.float32),
    )(v)
    return out


# speedup vs baseline: 1.3445x; 1.3445x over previous
import functools

import jax
import jax.numpy as jnp
from jax.experimental import pallas as pl
from jax.experimental.pallas import tpu as pltpu

NBUF = 4


def _fused_body(we_hbm, x_hbm, wg_ref, be_ref, su_ref, out_ref,
                webuf, xbuf, w4_ref, sem_we, sem_x, *, cap, t, f, kwe, c, seq):
    def we_copy(i):
        return pltpu.make_async_copy(
            we_hbm.at[i // (kwe // 2), pl.ds((i % (kwe // 2)) * f, f), :],
            webuf.at[i % NBUF], sem_we.at[i % NBUF])

    def x_copy(j):
        return pltpu.make_async_copy(
            x_hbm.at[pl.ds(j * t, t), :], xbuf.at[j % NBUF], sem_x.at[j % NBUF])

    for i in range(NBUF):
        we_copy(i).start()

    w4_ref[0:2] = jnp.transpose(wg_ref[...])
    w4_ref[2:4] = jnp.zeros_like(w4_ref[2:4])
    bs0 = jnp.sum(be_ref[0:1, :])
    bs1 = jnp.sum(be_ref[1:2, :])

    for i in range(kwe):
        we_copy(i).wait()
        part = jnp.sum(webuf[i % NBUF], axis=0, keepdims=True)
        w4_ref[pl.ds(2 + i // (kwe // 2), 1)] += part
        if i + NBUF < kwe:
            we_copy(i + NBUF).start()
        else:
            x_copy(i + NBUF - kwe).start()

    cnt = jnp.float32(0.0)
    cols = max(seq // t, 1)
    for j in range(c):
        x_copy(j).wait()
        lt = jax.lax.dot_general(
            w4_ref[...], xbuf[j % NBUF],
            dimension_numbers=(((1,), (1,)), ((), ())),
            preferred_element_type=jnp.float32)
        if j + NBUF < c:
            x_copy(j + NBUF).start()
        l0, l1, t0, t1 = lt[0:1], lt[1:2], lt[2:3], lt[3:4]
        e1 = l1 > l0
        m = e1.astype(jnp.float32)
        excl = jnp.dot(m, su_ref[...], preferred_element_type=jnp.float32)
        pos1 = excl + cnt
        slin = (j * t + jax.lax.broadcasted_iota(jnp.int32, (1, t), 1)
                ).astype(jnp.float32)
        pos0 = slin - pos1
        pos = jnp.where(e1, pos1, pos0)
        within = (pos < cap).astype(jnp.float32)
        gate = jax.nn.sigmoid(jnp.abs(l1 - l0))
        tsel = jnp.where(e1, t1 + bs1, t0 + bs0)
        r, col = j // cols, (j % cols) * t
        out_ref[r:r + 1, pl.ds(col, t)] = gate * within * tsel
        cnt = cnt + jnp.sum(m)

    for b in range(out_ref.shape[0]):
        row = out_ref[b:b + 1, :]
        mx = jnp.max(row, axis=1, keepdims=True)
        lse = jnp.log(jnp.sum(jnp.exp(row - mx), axis=1, keepdims=True)) + mx
        out_ref[b:b + 1, :] = row - lse


def kernel(input, wg, We, be):
    B, SEQ, D = input.shape
    E = wg.shape[1]
    S = B * SEQ
    cap = -(-S // E)

    x = input.reshape(S, D)

    F = 512
    KWE = E * (D // F)
    T = 512
    C = S // T

    ii = jax.lax.broadcasted_iota(jnp.int32, (T, T), 0)
    jj = jax.lax.broadcasted_iota(jnp.int32, (T, T), 1)
    su = (ii < jj).astype(jnp.float32)

    out = pl.pallas_call(
        functools.partial(_fused_body, cap=float(cap), t=T, f=F,
                          kwe=KWE, c=C, seq=SEQ),
        in_specs=[
            pl.BlockSpec(memory_space=pl.ANY),
            pl.BlockSpec(memory_space=pl.ANY),
            pl.BlockSpec((D, E), lambda: (0, 0)),
            pl.BlockSpec((E, D), lambda: (0, 0)),
            pl.BlockSpec((T, T), lambda: (0, 0)),
        ],
        out_specs=pl.BlockSpec((B, SEQ), lambda: (0, 0)),
        out_shape=jax.ShapeDtypeStruct((B, SEQ), jnp.float32),
        scratch_shapes=[
            pltpu.VMEM((NBUF, F, D), jnp.float32),
            pltpu.VMEM((NBUF, T, D), jnp.float32),
            pltpu.VMEM((4, D), jnp.float32),
            pltpu.SemaphoreType.DMA((NBUF,)),
            pltpu.SemaphoreType.DMA((NBUF,)),
        ],
    )(We, x, wg, be, su)
    return out
